# baseline (speedup 1.0000x reference)
import jax
import jax.numpy as jnp
from jax.experimental import pallas as pl

_BM = 400


def _pass1_kernel(x0_full_ref, l_ref, x0_blk_ref, w1_ref, w2_ref, y_ref):
    x1 = jnp.dot(l_ref[...], x0_full_ref[...], preferred_element_type=jnp.float32)
    y_ref[...] = (
        jnp.dot(x0_blk_ref[...], w1_ref[...], preferred_element_type=jnp.float32)
        + 2.0 * jnp.dot(x1, w2_ref[...], preferred_element_type=jnp.float32)
    )


def _pass2_kernel(y_full_ref, l_ref, x0_blk_ref, w02_ref, b_ref, out_ref):
    out_ref[...] = (
        jnp.dot(l_ref[...], y_full_ref[...], preferred_element_type=jnp.float32)
        + jnp.dot(x0_blk_ref[...], w02_ref[...], preferred_element_type=jnp.float32)
        + b_ref[...]
    )


def kernel(x, laplacian, weight, bias):
    B, Cin, V = x.shape
    K, _, Cout = weight.shape
    N = B * Cin

    x0 = x.reshape(N, V).T
    w0, w1, w2 = weight[0], weight[1], weight[2]
    w02 = w0 - w2
    b2 = bias.reshape(1, Cout)

    grid = (V // _BM,)
    full_spec = pl.BlockSpec((V, N), lambda j: (0, 0))
    l_spec = pl.BlockSpec((_BM, V), lambda j: (j, 0))
    blk_spec = pl.BlockSpec((_BM, N), lambda j: (j, 0))
    w_spec = pl.BlockSpec((Cin, Cout), lambda j: (0, 0))
    b_spec = pl.BlockSpec((1, Cout), lambda j: (0, 0))
    out_spec = pl.BlockSpec((_BM, Cout), lambda j: (j, 0))

    y = pl.pallas_call(
        _pass1_kernel,
        grid=grid,
        in_specs=[full_spec, l_spec, blk_spec, w_spec, w_spec],
        out_specs=out_spec,
        out_shape=jax.ShapeDtypeStruct((V, Cout), jnp.float32),
    )(x0, laplacian, x0, w1, w2)

    out = pl.pallas_call(
        _pass2_kernel,
        grid=grid,
        in_specs=[full_spec, l_spec, blk_spec, w_spec, b_spec],
        out_specs=out_spec,
        out_shape=jax.ShapeDtypeStruct((V, Cout), jnp.float32),
    )(y, laplacian, x0, w02, b2)

    return out.T.reshape(B, Cout, V)


# speedup vs baseline: 1.0652x; 1.0652x over previous
import jax
import jax.numpy as jnp
from jax.experimental import pallas as pl
from jax.experimental.pallas import tpu as pltpu

_BM = 400


def _fused_kernel(x0_ref, l_ref, w1_ref, w2_ref, w02_ref, b_ref, out_ref, y_ref):
    p = pl.program_id(0)
    j = pl.program_id(1)
    x0_blk = x0_ref[pl.ds(j * _BM, _BM), :]

    @pl.when(p == 0)
    def _phase0():
        x1 = jnp.dot(l_ref[...], x0_ref[...], preferred_element_type=jnp.float32)
        y_ref[pl.ds(j * _BM, _BM), :] = (
            jnp.dot(x0_blk, w1_ref[...], preferred_element_type=jnp.float32)
            + 2.0 * jnp.dot(x1, w2_ref[...], preferred_element_type=jnp.float32)
        )

    @pl.when(p == 1)
    def _phase1():
        out_ref[...] = (
            jnp.dot(l_ref[...], y_ref[...], preferred_element_type=jnp.float32)
            + jnp.dot(x0_blk, w02_ref[...], preferred_element_type=jnp.float32)
            + b_ref[...]
        )


def kernel(x, laplacian, weight, bias):
    B, Cin, V = x.shape
    K, _, Cout = weight.shape
    N = B * Cin

    x0 = x.reshape(N, V).T
    w0, w1, w2 = weight[0], weight[1], weight[2]
    w02 = w0 - w2
    b2 = bias.reshape(1, Cout)

    grid = (2, V // _BM)
    out = pl.pallas_call(
        _fused_kernel,
        grid=grid,
        in_specs=[
            pl.BlockSpec((V, N), lambda p, j: (0, 0)),
            pl.BlockSpec((_BM, V), lambda p, j: (j, 0)),
            pl.BlockSpec((Cin, Cout), lambda p, j: (0, 0)),
            pl.BlockSpec((Cin, Cout), lambda p, j: (0, 0)),
            pl.BlockSpec((Cin, Cout), lambda p, j: (0, 0)),
            pl.BlockSpec((1, Cout), lambda p, j: (0, 0)),
        ],
        out_specs=pl.BlockSpec((_BM, Cout), lambda p, j: (p * j, 0)),
        out_shape=jax.ShapeDtypeStruct((V, Cout), jnp.float32),
        scratch_shapes=[pltpu.VMEM((V, Cout), jnp.float32)],
    )(x0, laplacian, w1, w2, w02, b2)

    return out.T.reshape(B, Cout, V)
